# Initial kernel scaffold; baseline (speedup 1.0000x reference)
import jax
import jax.numpy as jnp
from jax.experimental import pallas as pl


def kernel(x, w_output):
    raise NotImplementedError("write your pallas kernel here")


# speedup vs baseline: 3.3676x; 3.3676x over previous
import functools

import jax
import jax.numpy as jnp
import numpy as np
from jax.experimental import pallas as pl
from jax.experimental.pallas import tpu as pltpu

CODE_LEN = 512
INFO_LEN = 256
DESIGN_SNR = 2.0
ITERS = 5
CLIP = 10.0
BATCH = 512
EPS = 1e-7

_INTERPRET = False
_PRECISION = jax.lax.Precision.HIGHEST


def _build_graph():
    n = int(np.log2(CODE_LEN))
    F = np.array([[1, 0], [1, 1]], dtype=np.int64)
    G = np.array([[1]], dtype=np.int64)
    for _ in range(n):
        G = np.kron(G, F)
    S = 10.0 ** (DESIGN_SNR / 10.0)
    z = np.array([np.exp(-S)], dtype=np.float64)
    while z.size < CODE_LEN:
        z = np.concatenate([2.0 * z - z ** 2, z ** 2])
    order = np.argsort(z, kind='stable')
    info = np.zeros(CODE_LEN, dtype=bool)
    info[order[:INFO_LEN]] = True
    pcm = G[:, ~info].T.astype(np.float32)
    chk, var = np.nonzero(pcm)
    return info, pcm, chk.astype(np.int32), var.astype(np.int32)


_INFO_NP, _PCM_NP, _CHK_NP, _VAR_NP = _build_graph()
_E = int(_CHK_NP.shape[0])
_NCHK = int(_PCM_NP.shape[0])

_EB = 1024
_NEB = (_E + _EB - 1) // _EB
_EP = _NEB * _EB

_MV_NP = np.zeros((_EP, CODE_LEN), dtype=np.float32)
_MV_NP[np.arange(_E), _VAR_NP] = 1.0
_MC_NP = np.zeros((_EP, _NCHK), dtype=np.float32)
_MC_NP[np.arange(_E), _CHK_NP] = 1.0

_MV = jnp.asarray(_MV_NP)
_MC = jnp.asarray(_MC_NP)
_PCM = jnp.asarray(_PCM_NP)
_INFO_IDX = jnp.asarray(np.nonzero(_INFO_NP)[0].astype(np.int32))

_B = BATCH


def _dot(a, b):
    return jax.lax.dot_general(a, b, (((1,), (0,)), ((), ())),
                               precision=_PRECISION,
                               preferred_element_type=jnp.float32)


def _dott(a, b):
    return jax.lax.dot_general(a, b, (((0,), (0,)), ((), ())),
                               precision=_PRECISION,
                               preferred_element_type=jnp.float32)


def _edge_metric(pre):
    t = jnp.tanh(0.5 * pre)
    la = jnp.log(jnp.abs(t) + 1e-12)
    ng = (t < 0).astype(jnp.float32)
    return la, ng


def _stage_a_body(*refs, first):
    if first:
        node_ref, mv_ref, mc_ref, sl_ref, sc_ref, nc_ref = refs
        even_ref = None
    else:
        node_ref, even_ref, mv_ref, mc_ref, sl_ref, sc_ref, nc_ref = refs
    g = _dot(mv_ref[...], node_ref[...])
    if first:
        pre = jnp.clip(g, -CLIP, CLIP)
    else:
        pre = jnp.clip(g - even_ref[...], -CLIP, CLIP)
    la, ng = _edge_metric(pre)
    sl_ref[...] = la * (1.0 - 2.0 * ng)

    @pl.when(pl.program_id(0) == 0)
    def _():
        sc_ref[...] = jnp.zeros_like(sc_ref)
        nc_ref[...] = jnp.zeros_like(nc_ref)

    sc_ref[...] += _dott(mc_ref[...], la)
    nc_ref[...] += _dott(mc_ref[...], ng)


def _stage_a(node, even, first):
    body = functools.partial(_stage_a_body, first=first)
    in_specs = [pl.BlockSpec((CODE_LEN, _B), lambda i: (0, 0))]
    args = [node]
    if not first:
        in_specs.append(pl.BlockSpec((_EB, _B), lambda i: (i, 0)))
        args.append(even)
    in_specs += [
        pl.BlockSpec((_EB, CODE_LEN), lambda i: (i, 0)),
        pl.BlockSpec((_EB, _NCHK), lambda i: (i, 0)),
    ]
    args += [_MV, _MC]
    out_specs = [
        pl.BlockSpec((_EB, _B), lambda i: (i, 0)),
        pl.BlockSpec((_NCHK, _B), lambda i: (0, 0)),
        pl.BlockSpec((_NCHK, _B), lambda i: (0, 0)),
    ]
    out_shape = [
        jax.ShapeDtypeStruct((_EP, _B), jnp.float32),
        jax.ShapeDtypeStruct((_NCHK, _B), jnp.float32),
        jax.ShapeDtypeStruct((_NCHK, _B), jnp.float32),
    ]
    return pl.pallas_call(
        body, grid=(_NEB,), in_specs=in_specs, out_specs=out_specs,
        out_shape=out_shape, interpret=_INTERPRET)(*args)


def _stage_b_body(*refs, first):
    if first:
        sl_ref, sc_ref, nc_ref, mc_ref, mv_ref, even_ref, tot_ref = refs
        even_old_ref = amask_ref = None
    else:
        (sl_ref, sc_ref, nc_ref, mc_ref, mv_ref, even_old_ref,
         amask_ref, even_ref, tot_ref) = refs
    sl = sl_ref[...]
    la = -jnp.abs(sl)
    ng = (sl > 0).astype(jnp.float32)
    se = _dot(mc_ref[...], sc_ref[...]) - la
    ne = _dot(mc_ref[...], nc_ref[...]) - ng
    sign = 1.0 - 2.0 * jnp.mod(ne, 2.0)
    prod = jnp.clip(sign * jnp.exp(se), -1.0 + EPS, 1.0 - EPS)
    ev_new = jnp.log((1.0 + prod) / (1.0 - prod))
    if first:
        ev = ev_new
    else:
        a = amask_ref[0:1, :]
        ev = a * ev_new + (1.0 - a) * even_old_ref[...]
    even_ref[...] = ev

    @pl.when(pl.program_id(0) == 0)
    def _():
        tot_ref[...] = jnp.zeros_like(tot_ref)

    tot_ref[...] += _dott(mv_ref[...], ev)


def _stage_b(sl, sc, nc, even_old, amask, first):
    body = functools.partial(_stage_b_body, first=first)
    in_specs = [
        pl.BlockSpec((_EB, _B), lambda i: (i, 0)),
        pl.BlockSpec((_NCHK, _B), lambda i: (0, 0)),
        pl.BlockSpec((_NCHK, _B), lambda i: (0, 0)),
        pl.BlockSpec((_EB, _NCHK), lambda i: (i, 0)),
        pl.BlockSpec((_EB, CODE_LEN), lambda i: (i, 0)),
    ]
    args = [sl, sc, nc, _MC, _MV]
    if not first:
        in_specs += [
            pl.BlockSpec((_EB, _B), lambda i: (i, 0)),
            pl.BlockSpec((8, _B), lambda i: (0, 0)),
        ]
        args += [even_old, amask]
    out_specs = [
        pl.BlockSpec((_EB, _B), lambda i: (i, 0)),
        pl.BlockSpec((CODE_LEN, _B), lambda i: (0, 0)),
    ]
    out_shape = [
        jax.ShapeDtypeStruct((_EP, _B), jnp.float32),
        jax.ShapeDtypeStruct((CODE_LEN, _B), jnp.float32),
    ]
    return pl.pallas_call(
        body, grid=(_NEB,), in_specs=in_specs, out_specs=out_specs,
        out_shape=out_shape, interpret=_INTERPRET)(*args)


def _stage_c_body(xt_ref, tot_ref, amask_ref, of_ref, pcm_ref,
                  out_ns_ref, of_new_ref, amask_new_ref):
    out_ns = xt_ref[...] + tot_ref[...]
    out_ns_ref[...] = out_ns
    a = amask_ref[0:1, :]
    of_new = a * out_ns + (1.0 - a) * of_ref[...]
    of_new_ref[...] = of_new
    bits = (out_ns < 0).astype(jnp.float32)
    syn = _dot(pcm_ref[...], bits)
    par = syn - 2.0 * jnp.floor(0.5 * syn)
    bad = jnp.max(par, axis=0, keepdims=True)
    okf = (bad < 0.5).astype(jnp.float32)
    a_new = a * (1.0 - okf)
    amask_new_ref[...] = jnp.broadcast_to(a_new, amask_new_ref.shape)


def _stage_c(xt, tot, amask, out_final):
    out_shape = [
        jax.ShapeDtypeStruct((CODE_LEN, _B), jnp.float32),
        jax.ShapeDtypeStruct((CODE_LEN, _B), jnp.float32),
        jax.ShapeDtypeStruct((8, _B), jnp.float32),
    ]
    return pl.pallas_call(
        _stage_c_body, out_shape=out_shape,
        interpret=_INTERPRET)(xt, tot, amask, out_final, _PCM)


def _stage_c0_body(xt_ref, tot_ref, out_ns_ref):
    out_ns_ref[...] = xt_ref[...] + tot_ref[...]


def _stage_c0(xt, tot):
    return pl.pallas_call(
        _stage_c0_body,
        out_shape=jax.ShapeDtypeStruct((CODE_LEN, _B), jnp.float32),
        interpret=_INTERPRET)(xt, tot)


def _stage_d_body(even_ref, w_ref, xt_ref, amask_ref, of_ref,
                  out5_ref, acc_ref):
    @pl.when(pl.program_id(0) == 0)
    def _():
        acc_ref[...] = jnp.zeros_like(acc_ref)

    acc_ref[...] += _dott(w_ref[...], even_ref[...])

    @pl.when(pl.program_id(0) == _NEB - 1)
    def _():
        a = amask_ref[0:1, :]
        out5_ref[...] = a * (xt_ref[...] + acc_ref[...]) \
            + (1.0 - a) * of_ref[...]


def _stage_d(even, w_pad, xt, amask, out_final):
    in_specs = [
        pl.BlockSpec((_EB, _B), lambda i: (i, 0)),
        pl.BlockSpec((_EB, CODE_LEN), lambda i: (i, 0)),
        pl.BlockSpec((CODE_LEN, _B), lambda i: (0, 0)),
        pl.BlockSpec((8, _B), lambda i: (0, 0)),
        pl.BlockSpec((CODE_LEN, _B), lambda i: (0, 0)),
    ]
    return pl.pallas_call(
        _stage_d_body, grid=(_NEB,), in_specs=in_specs,
        out_specs=pl.BlockSpec((CODE_LEN, _B), lambda i: (0, 0)),
        out_shape=jax.ShapeDtypeStruct((CODE_LEN, _B), jnp.float32),
        scratch_shapes=[pltpu.VMEM((CODE_LEN, _B), jnp.float32)],
        interpret=_INTERPRET)(even, w_pad, xt, amask, out_final)


def kernel(x, w_output):
    xt = x.T
    w_pad = jnp.pad(w_output, ((0, _EP - _E), (0, 0)))
    ones_a = jnp.ones((8, _B), jnp.float32)

    outs = []
    sl, sc, nc = _stage_a(xt, None, first=True)
    even, tot = _stage_b(sl, sc, nc, None, None, first=True)
    out0 = _stage_c0(xt, tot)
    outs.append(out0)

    amask = ones_a
    out_final = jnp.zeros((CODE_LEN, _B), jnp.float32)
    node = out0
    for _ in range(ITERS - 1):
        sl, sc, nc = _stage_a(node, even, first=False)
        even, tot = _stage_b(sl, sc, nc, even, amask, first=False)
        out_ns, out_final, amask = _stage_c(xt, tot, amask, out_final)
        outs.append(out_ns)
        node = out_ns

    outs.append(_stage_d(even, w_pad, xt, amask, out_final))
    return tuple(o.T[:, _INFO_IDX] for o in outs)


# speedup vs baseline: 8.3140x; 2.4688x over previous
import functools

import jax
import jax.numpy as jnp
import numpy as np
from jax.experimental import pallas as pl
from jax.experimental.pallas import tpu as pltpu

CODE_LEN = 512
INFO_LEN = 256
DESIGN_SNR = 2.0
ITERS = 5
CLIP = 10.0
BATCH = 512
EPS = 1e-7

_INTERPRET = False
_PRECISION = jax.lax.Precision.DEFAULT


def _build_graph():
    n = int(np.log2(CODE_LEN))
    F = np.array([[1, 0], [1, 1]], dtype=np.int64)
    G = np.array([[1]], dtype=np.int64)
    for _ in range(n):
        G = np.kron(G, F)
    S = 10.0 ** (DESIGN_SNR / 10.0)
    z = np.array([np.exp(-S)], dtype=np.float64)
    while z.size < CODE_LEN:
        z = np.concatenate([2.0 * z - z ** 2, z ** 2])
    order = np.argsort(z, kind='stable')
    info = np.zeros(CODE_LEN, dtype=bool)
    info[order[:INFO_LEN]] = True
    pcm = G[:, ~info].T.astype(np.float32)
    chk, var = np.nonzero(pcm)
    return info, pcm, chk.astype(np.int32), var.astype(np.int32)


_INFO_NP, _PCM_NP, _CHK_NP, _VAR_NP = _build_graph()
_E = int(_CHK_NP.shape[0])
_NCHK = int(_PCM_NP.shape[0])

_EB = 1024
_NEB = (_E + _EB - 1) // _EB
_EP = _NEB * _EB

_MV_NP = np.zeros((_EP, CODE_LEN), dtype=np.float32)
_MV_NP[np.arange(_E), _VAR_NP] = 1.0
_MC_NP = np.zeros((_EP, _NCHK), dtype=np.float32)
_MC_NP[np.arange(_E), _CHK_NP] = 1.0

_MV = jnp.asarray(_MV_NP)
_MC = jnp.asarray(_MC_NP)
_PCM = jnp.asarray(_PCM_NP)
_INFO_IDX = jnp.asarray(np.nonzero(_INFO_NP)[0].astype(np.int32))

_B = BATCH


def _dot(a, b):
    return jax.lax.dot_general(a, b, (((1,), (0,)), ((), ())),
                               precision=_PRECISION,
                               preferred_element_type=jnp.float32)


def _dott(a, b):
    return jax.lax.dot_general(a, b, (((0,), (0,)), ((), ())),
                               precision=_PRECISION,
                               preferred_element_type=jnp.float32)


def _edge_metric(pre):
    t = jnp.tanh(0.5 * pre)
    la = jnp.log(jnp.abs(t) + 1e-12)
    ng = (t < 0).astype(jnp.float32)
    return la, ng


def _stage_a_body(*refs, first):
    if first:
        node_ref, mv_ref, mc_ref, sl_ref, sc_ref, nc_ref = refs
        even_ref = None
    else:
        node_ref, even_ref, mv_ref, mc_ref, sl_ref, sc_ref, nc_ref = refs
    g = _dot(mv_ref[...], node_ref[...])
    if first:
        pre = jnp.clip(g, -CLIP, CLIP)
    else:
        pre = jnp.clip(g - even_ref[...], -CLIP, CLIP)
    la, ng = _edge_metric(pre)
    sl_ref[...] = la * (1.0 - 2.0 * ng)

    @pl.when(pl.program_id(0) == 0)
    def _():
        sc_ref[...] = jnp.zeros_like(sc_ref)
        nc_ref[...] = jnp.zeros_like(nc_ref)

    sc_ref[...] += _dott(mc_ref[...], la)
    nc_ref[...] += _dott(mc_ref[...], ng)


def _stage_a(node, even, first):
    body = functools.partial(_stage_a_body, first=first)
    in_specs = [pl.BlockSpec((CODE_LEN, _B), lambda i: (0, 0))]
    args = [node]
    if not first:
        in_specs.append(pl.BlockSpec((_EB, _B), lambda i: (i, 0)))
        args.append(even)
    in_specs += [
        pl.BlockSpec((_EB, CODE_LEN), lambda i: (i, 0)),
        pl.BlockSpec((_EB, _NCHK), lambda i: (i, 0)),
    ]
    args += [_MV, _MC]
    out_specs = [
        pl.BlockSpec((_EB, _B), lambda i: (i, 0)),
        pl.BlockSpec((_NCHK, _B), lambda i: (0, 0)),
        pl.BlockSpec((_NCHK, _B), lambda i: (0, 0)),
    ]
    out_shape = [
        jax.ShapeDtypeStruct((_EP, _B), jnp.float32),
        jax.ShapeDtypeStruct((_NCHK, _B), jnp.float32),
        jax.ShapeDtypeStruct((_NCHK, _B), jnp.float32),
    ]
    return pl.pallas_call(
        body, grid=(_NEB,), in_specs=in_specs, out_specs=out_specs,
        out_shape=out_shape, interpret=_INTERPRET)(*args)


def _stage_b_body(*refs, first):
    if first:
        sl_ref, sc_ref, nc_ref, mc_ref, mv_ref, even_ref, tot_ref = refs
        even_old_ref = amask_ref = None
    else:
        (sl_ref, sc_ref, nc_ref, mc_ref, mv_ref, even_old_ref,
         amask_ref, even_ref, tot_ref) = refs
    sl = sl_ref[...]
    la = -jnp.abs(sl)
    ng = (sl > 0).astype(jnp.float32)
    se = _dot(mc_ref[...], sc_ref[...]) - la
    ne = _dot(mc_ref[...], nc_ref[...]) - ng
    sign = 1.0 - 2.0 * jnp.mod(ne, 2.0)
    prod = jnp.clip(sign * jnp.exp(se), -1.0 + EPS, 1.0 - EPS)
    ev_new = jnp.log((1.0 + prod) / (1.0 - prod))
    if first:
        ev = ev_new
    else:
        a = amask_ref[0:1, :]
        ev = a * ev_new + (1.0 - a) * even_old_ref[...]
    even_ref[...] = ev

    @pl.when(pl.program_id(0) == 0)
    def _():
        tot_ref[...] = jnp.zeros_like(tot_ref)

    tot_ref[...] += _dott(mv_ref[...], ev)


def _stage_b(sl, sc, nc, even_old, amask, first):
    body = functools.partial(_stage_b_body, first=first)
    in_specs = [
        pl.BlockSpec((_EB, _B), lambda i: (i, 0)),
        pl.BlockSpec((_NCHK, _B), lambda i: (0, 0)),
        pl.BlockSpec((_NCHK, _B), lambda i: (0, 0)),
        pl.BlockSpec((_EB, _NCHK), lambda i: (i, 0)),
        pl.BlockSpec((_EB, CODE_LEN), lambda i: (i, 0)),
    ]
    args = [sl, sc, nc, _MC, _MV]
    if not first:
        in_specs += [
            pl.BlockSpec((_EB, _B), lambda i: (i, 0)),
            pl.BlockSpec((8, _B), lambda i: (0, 0)),
        ]
        args += [even_old, amask]
    out_specs = [
        pl.BlockSpec((_EB, _B), lambda i: (i, 0)),
        pl.BlockSpec((CODE_LEN, _B), lambda i: (0, 0)),
    ]
    out_shape = [
        jax.ShapeDtypeStruct((_EP, _B), jnp.float32),
        jax.ShapeDtypeStruct((CODE_LEN, _B), jnp.float32),
    ]
    return pl.pallas_call(
        body, grid=(_NEB,), in_specs=in_specs, out_specs=out_specs,
        out_shape=out_shape, interpret=_INTERPRET)(*args)


def _stage_c_body(xt_ref, tot_ref, amask_ref, of_ref, pcm_ref,
                  out_ns_ref, of_new_ref, amask_new_ref):
    out_ns = xt_ref[...] + tot_ref[...]
    out_ns_ref[...] = out_ns
    a = amask_ref[0:1, :]
    of_new = a * out_ns + (1.0 - a) * of_ref[...]
    of_new_ref[...] = of_new
    bits = (out_ns < 0).astype(jnp.float32)
    syn = _dot(pcm_ref[...], bits)
    par = syn - 2.0 * jnp.floor(0.5 * syn)
    bad = jnp.max(par, axis=0, keepdims=True)
    okf = (bad < 0.5).astype(jnp.float32)
    a_new = a * (1.0 - okf)
    amask_new_ref[...] = jnp.broadcast_to(a_new, amask_new_ref.shape)


def _stage_c(xt, tot, amask, out_final):
    out_shape = [
        jax.ShapeDtypeStruct((CODE_LEN, _B), jnp.float32),
        jax.ShapeDtypeStruct((CODE_LEN, _B), jnp.float32),
        jax.ShapeDtypeStruct((8, _B), jnp.float32),
    ]
    return pl.pallas_call(
        _stage_c_body, out_shape=out_shape,
        interpret=_INTERPRET)(xt, tot, amask, out_final, _PCM)


def _stage_c0_body(xt_ref, tot_ref, out_ns_ref):
    out_ns_ref[...] = xt_ref[...] + tot_ref[...]


def _stage_c0(xt, tot):
    return pl.pallas_call(
        _stage_c0_body,
        out_shape=jax.ShapeDtypeStruct((CODE_LEN, _B), jnp.float32),
        interpret=_INTERPRET)(xt, tot)


def _stage_d_body(even_ref, w_ref, xt_ref, amask_ref, of_ref,
                  out5_ref, acc_ref):
    @pl.when(pl.program_id(0) == 0)
    def _():
        acc_ref[...] = jnp.zeros_like(acc_ref)

    acc_ref[...] += _dott(w_ref[...], even_ref[...])

    @pl.when(pl.program_id(0) == _NEB - 1)
    def _():
        a = amask_ref[0:1, :]
        out5_ref[...] = a * (xt_ref[...] + acc_ref[...]) \
            + (1.0 - a) * of_ref[...]


def _stage_d(even, w_pad, xt, amask, out_final):
    in_specs = [
        pl.BlockSpec((_EB, _B), lambda i: (i, 0)),
        pl.BlockSpec((_EB, CODE_LEN), lambda i: (i, 0)),
        pl.BlockSpec((CODE_LEN, _B), lambda i: (0, 0)),
        pl.BlockSpec((8, _B), lambda i: (0, 0)),
        pl.BlockSpec((CODE_LEN, _B), lambda i: (0, 0)),
    ]
    return pl.pallas_call(
        _stage_d_body, grid=(_NEB,), in_specs=in_specs,
        out_specs=pl.BlockSpec((CODE_LEN, _B), lambda i: (0, 0)),
        out_shape=jax.ShapeDtypeStruct((CODE_LEN, _B), jnp.float32),
        scratch_shapes=[pltpu.VMEM((CODE_LEN, _B), jnp.float32)],
        interpret=_INTERPRET)(even, w_pad, xt, amask, out_final)


def kernel(x, w_output):
    xt = x.T
    w_pad = jnp.pad(w_output, ((0, _EP - _E), (0, 0)))
    ones_a = jnp.ones((8, _B), jnp.float32)

    outs = []
    sl, sc, nc = _stage_a(xt, None, first=True)
    even, tot = _stage_b(sl, sc, nc, None, None, first=True)
    out0 = _stage_c0(xt, tot)
    outs.append(out0)

    amask = ones_a
    out_final = jnp.zeros((CODE_LEN, _B), jnp.float32)
    node = out0
    for _ in range(ITERS - 1):
        sl, sc, nc = _stage_a(node, even, first=False)
        even, tot = _stage_b(sl, sc, nc, even, amask, first=False)
        out_ns, out_final, amask = _stage_c(xt, tot, amask, out_final)
        outs.append(out_ns)
        node = out_ns

    outs.append(_stage_d(even, w_pad, xt, amask, out_final))
    return tuple(o.T[:, _INFO_IDX] for o in outs)


# speedup vs baseline: 8.9106x; 1.0718x over previous
import functools

import jax
import jax.numpy as jnp
import numpy as np
from jax.experimental import pallas as pl
from jax.experimental.pallas import tpu as pltpu

CODE_LEN = 512
INFO_LEN = 256
DESIGN_SNR = 2.0
ITERS = 5
CLIP = 10.0
BATCH = 512
EPS = 1e-7

_INTERPRET = False
_PRECISION = jax.lax.Precision.DEFAULT


def _build_graph():
    n = int(np.log2(CODE_LEN))
    F = np.array([[1, 0], [1, 1]], dtype=np.int64)
    G = np.array([[1]], dtype=np.int64)
    for _ in range(n):
        G = np.kron(G, F)
    S = 10.0 ** (DESIGN_SNR / 10.0)
    z = np.array([np.exp(-S)], dtype=np.float64)
    while z.size < CODE_LEN:
        z = np.concatenate([2.0 * z - z ** 2, z ** 2])
    order = np.argsort(z, kind='stable')
    info = np.zeros(CODE_LEN, dtype=bool)
    info[order[:INFO_LEN]] = True
    pcm = G[:, ~info].T.astype(np.float32)
    chk, var = np.nonzero(pcm)
    return info, pcm, chk.astype(np.int32), var.astype(np.int32)


_INFO_NP, _PCM_NP, _CHK_NP, _VAR_NP = _build_graph()
_E = int(_CHK_NP.shape[0])
_NCHK = int(_PCM_NP.shape[0])

_EB = 1024
_NEB = (_E + _EB - 1) // _EB
_EP = _NEB * _EB

_MV_NP = np.zeros((_EP, CODE_LEN), dtype=np.float32)
_MV_NP[np.arange(_E), _VAR_NP] = 1.0
_MC_NP = np.zeros((_EP, _NCHK), dtype=np.float32)
_MC_NP[np.arange(_E), _CHK_NP] = 1.0

_MV = jnp.asarray(_MV_NP, dtype=jnp.bfloat16)
_MC = jnp.asarray(_MC_NP, dtype=jnp.bfloat16)
_PCM = jnp.asarray(_PCM_NP)
_INFO_IDX = jnp.asarray(np.nonzero(_INFO_NP)[0].astype(np.int32))

_B = BATCH


def _dot(a, b):
    return jax.lax.dot_general(a, b, (((1,), (0,)), ((), ())),
                               precision=_PRECISION,
                               preferred_element_type=jnp.float32)


def _dott(a, b):
    return jax.lax.dot_general(a, b, (((0,), (0,)), ((), ())),
                               precision=_PRECISION,
                               preferred_element_type=jnp.float32)


def _edge_metric(pre):
    t = jnp.tanh(0.5 * pre)
    la = jnp.log(jnp.abs(t) + 1e-12)
    ng = (t < 0).astype(jnp.float32)
    return la, ng


def _stage_a_body(*refs, first):
    if first:
        node_ref, mv_ref, mc_ref, sl_ref, sc_ref, nc_ref = refs
        even_ref = None
    else:
        node_ref, even_ref, mv_ref, mc_ref, sl_ref, sc_ref, nc_ref = refs
    g = _dot(mv_ref[...], node_ref[...])
    if first:
        pre = jnp.clip(g, -CLIP, CLIP)
    else:
        pre = jnp.clip(g - even_ref[...], -CLIP, CLIP)
    la, ng = _edge_metric(pre)
    sl_ref[...] = la * (1.0 - 2.0 * ng)

    @pl.when(pl.program_id(0) == 0)
    def _():
        sc_ref[...] = jnp.zeros_like(sc_ref)
        nc_ref[...] = jnp.zeros_like(nc_ref)

    sc_ref[...] += _dott(mc_ref[...], la)
    nc_ref[...] += _dott(mc_ref[...], ng)


def _stage_a(node, even, first):
    body = functools.partial(_stage_a_body, first=first)
    in_specs = [pl.BlockSpec((CODE_LEN, _B), lambda i: (0, 0))]
    args = [node]
    if not first:
        in_specs.append(pl.BlockSpec((_EB, _B), lambda i: (i, 0)))
        args.append(even)
    in_specs += [
        pl.BlockSpec((_EB, CODE_LEN), lambda i: (i, 0)),
        pl.BlockSpec((_EB, _NCHK), lambda i: (i, 0)),
    ]
    args += [_MV, _MC]
    out_specs = [
        pl.BlockSpec((_EB, _B), lambda i: (i, 0)),
        pl.BlockSpec((_NCHK, _B), lambda i: (0, 0)),
        pl.BlockSpec((_NCHK, _B), lambda i: (0, 0)),
    ]
    out_shape = [
        jax.ShapeDtypeStruct((_EP, _B), jnp.float32),
        jax.ShapeDtypeStruct((_NCHK, _B), jnp.float32),
        jax.ShapeDtypeStruct((_NCHK, _B), jnp.float32),
    ]
    return pl.pallas_call(
        body, grid=(_NEB,), in_specs=in_specs, out_specs=out_specs,
        out_shape=out_shape, interpret=_INTERPRET)(*args)


def _stage_b_body(*refs, first):
    if first:
        sl_ref, sc_ref, nc_ref, mc_ref, mv_ref, even_ref, tot_ref = refs
        even_old_ref = amask_ref = None
    else:
        (sl_ref, sc_ref, nc_ref, mc_ref, mv_ref, even_old_ref,
         amask_ref, even_ref, tot_ref) = refs
    sl = sl_ref[...]
    la = -jnp.abs(sl)
    ng = (sl > 0).astype(jnp.float32)
    se = _dot(mc_ref[...], sc_ref[...]) - la
    ne = _dot(mc_ref[...], nc_ref[...]) - ng
    sign = 1.0 - 2.0 * jnp.mod(ne, 2.0)
    prod = jnp.clip(sign * jnp.exp(se), -1.0 + EPS, 1.0 - EPS)
    ev_new = jnp.log((1.0 + prod) / (1.0 - prod))
    if first:
        ev = ev_new
    else:
        a = amask_ref[0:1, :]
        ev = a * ev_new + (1.0 - a) * even_old_ref[...]
    even_ref[...] = ev

    @pl.when(pl.program_id(0) == 0)
    def _():
        tot_ref[...] = jnp.zeros_like(tot_ref)

    tot_ref[...] += _dott(mv_ref[...], ev)


def _stage_b(sl, sc, nc, even_old, amask, first):
    body = functools.partial(_stage_b_body, first=first)
    in_specs = [
        pl.BlockSpec((_EB, _B), lambda i: (i, 0)),
        pl.BlockSpec((_NCHK, _B), lambda i: (0, 0)),
        pl.BlockSpec((_NCHK, _B), lambda i: (0, 0)),
        pl.BlockSpec((_EB, _NCHK), lambda i: (i, 0)),
        pl.BlockSpec((_EB, CODE_LEN), lambda i: (i, 0)),
    ]
    args = [sl, sc, nc, _MC, _MV]
    if not first:
        in_specs += [
            pl.BlockSpec((_EB, _B), lambda i: (i, 0)),
            pl.BlockSpec((8, _B), lambda i: (0, 0)),
        ]
        args += [even_old, amask]
    out_specs = [
        pl.BlockSpec((_EB, _B), lambda i: (i, 0)),
        pl.BlockSpec((CODE_LEN, _B), lambda i: (0, 0)),
    ]
    out_shape = [
        jax.ShapeDtypeStruct((_EP, _B), jnp.float32),
        jax.ShapeDtypeStruct((CODE_LEN, _B), jnp.float32),
    ]
    return pl.pallas_call(
        body, grid=(_NEB,), in_specs=in_specs, out_specs=out_specs,
        out_shape=out_shape, interpret=_INTERPRET)(*args)


def _stage_c_body(xt_ref, tot_ref, amask_ref, of_ref, pcm_ref,
                  out_ns_ref, of_new_ref, amask_new_ref):
    out_ns = xt_ref[...] + tot_ref[...]
    out_ns_ref[...] = out_ns
    a = amask_ref[0:1, :]
    of_new = a * out_ns + (1.0 - a) * of_ref[...]
    of_new_ref[...] = of_new
    bits = (out_ns < 0).astype(jnp.float32)
    syn = _dot(pcm_ref[...], bits)
    par = syn - 2.0 * jnp.floor(0.5 * syn)
    bad = jnp.max(par, axis=0, keepdims=True)
    okf = (bad < 0.5).astype(jnp.float32)
    a_new = a * (1.0 - okf)
    amask_new_ref[...] = jnp.broadcast_to(a_new, amask_new_ref.shape)


def _stage_c(xt, tot, amask, out_final):
    out_shape = [
        jax.ShapeDtypeStruct((CODE_LEN, _B), jnp.float32),
        jax.ShapeDtypeStruct((CODE_LEN, _B), jnp.float32),
        jax.ShapeDtypeStruct((8, _B), jnp.float32),
    ]
    return pl.pallas_call(
        _stage_c_body, out_shape=out_shape,
        interpret=_INTERPRET)(xt, tot, amask, out_final, _PCM)


def _stage_c0_body(xt_ref, tot_ref, out_ns_ref):
    out_ns_ref[...] = xt_ref[...] + tot_ref[...]


def _stage_c0(xt, tot):
    return pl.pallas_call(
        _stage_c0_body,
        out_shape=jax.ShapeDtypeStruct((CODE_LEN, _B), jnp.float32),
        interpret=_INTERPRET)(xt, tot)


def _stage_d_body(even_ref, w_ref, xt_ref, amask_ref, of_ref,
                  out5_ref, acc_ref):
    @pl.when(pl.program_id(0) == 0)
    def _():
        acc_ref[...] = jnp.zeros_like(acc_ref)

    acc_ref[...] += _dott(w_ref[...], even_ref[...])

    @pl.when(pl.program_id(0) == _NEB - 1)
    def _():
        a = amask_ref[0:1, :]
        out5_ref[...] = a * (xt_ref[...] + acc_ref[...]) \
            + (1.0 - a) * of_ref[...]


def _stage_d(even, w_pad, xt, amask, out_final):
    in_specs = [
        pl.BlockSpec((_EB, _B), lambda i: (i, 0)),
        pl.BlockSpec((_EB, CODE_LEN), lambda i: (i, 0)),
        pl.BlockSpec((CODE_LEN, _B), lambda i: (0, 0)),
        pl.BlockSpec((8, _B), lambda i: (0, 0)),
        pl.BlockSpec((CODE_LEN, _B), lambda i: (0, 0)),
    ]
    return pl.pallas_call(
        _stage_d_body, grid=(_NEB,), in_specs=in_specs,
        out_specs=pl.BlockSpec((CODE_LEN, _B), lambda i: (0, 0)),
        out_shape=jax.ShapeDtypeStruct((CODE_LEN, _B), jnp.float32),
        scratch_shapes=[pltpu.VMEM((CODE_LEN, _B), jnp.float32)],
        interpret=_INTERPRET)(even, w_pad, xt, amask, out_final)


def kernel(x, w_output):
    xt = x.T
    w_pad = jnp.pad(w_output, ((0, _EP - _E), (0, 0)))
    ones_a = jnp.ones((8, _B), jnp.float32)

    outs = []
    sl, sc, nc = _stage_a(xt, None, first=True)
    even, tot = _stage_b(sl, sc, nc, None, None, first=True)
    out0 = _stage_c0(xt, tot)
    outs.append(out0)

    amask = ones_a
    out_final = jnp.zeros((CODE_LEN, _B), jnp.float32)
    node = out0
    for _ in range(ITERS - 1):
        sl, sc, nc = _stage_a(node, even, first=False)
        even, tot = _stage_b(sl, sc, nc, even, amask, first=False)
        out_ns, out_final, amask = _stage_c(xt, tot, amask, out_final)
        outs.append(out_ns)
        node = out_ns

    outs.append(_stage_d(even, w_pad, xt, amask, out_final))
    return tuple(o.T[:, _INFO_IDX] for o in outs)
